# baseline (speedup 1.0000x reference)
import functools

import jax
import jax.numpy as jnp
from jax import lax
from jax.experimental import pallas as pl
from jax.experimental.pallas import tpu as pltpu
from jax.experimental.pallas import tpu_sc as plsc

B, C, H, W = 4, 256, 200, 176
HW = H * W
N = 100000
BLK = 192
NBLK = (N + BLK - 1) // BLK
LAST_BASE = N - BLK
NW = 32
NITER = ((NBLK + NW - 1) // NW + 1) // 2


def _make_gather():
    mesh = plsc.VectorSubcoreMesh(core_axis_name="c", subcore_axis_name="s")

    @functools.partial(
        pl.kernel,
        mesh=mesh,
        out_type=jax.ShapeDtypeStruct((N, C), jnp.float32),
        scratch_types=[
            pltpu.VMEM((BLK,), jnp.int32), pltpu.VMEM((BLK,), jnp.int32),
            pltpu.VMEM((BLK,), jnp.int32), pltpu.VMEM((BLK,), jnp.int32),
            pltpu.VMEM((BLK,), jnp.int32), pltpu.VMEM((BLK,), jnp.int32),
            pltpu.VMEM((BLK,), jnp.int32), pltpu.VMEM((BLK,), jnp.int32),
            pltpu.VMEM((BLK, C), jnp.float32),
            pltpu.VMEM((BLK, C), jnp.float32),
            pltpu.SemaphoreType.DMA, pltpu.SemaphoreType.DMA,
            pltpu.SemaphoreType.DMA, pltpu.SemaphoreType.DMA,
            pltpu.SemaphoreType.DMA, pltpu.SemaphoreType.DMA,
        ],
    )
    def gather_kernel(table_hbm, b_hbm, y_hbm, x_hbm, out_hbm,
                      bv0, yv0, xv0, bv1, yv1, xv1, idx0, idx1,
                      rows0, rows1,
                      sem_c0, sem_c1, sem_g0, sem_g1, sem_w0, sem_w1):
        wid = lax.axis_index("s") * 2 + lax.axis_index("c")
        bv, yv, xv = (bv0, bv1), (yv0, yv1), (xv0, xv1)
        idx, rows = (idx0, idx1), (rows0, rows1)
        sem_c, sem_g, sem_w = (sem_c0, sem_c1), (sem_g0, sem_g1), (sem_w0, sem_w1)

        def blk_of(jj, s):
            g = wid + NW * (2 * jj + s)
            base = pl.multiple_of(jnp.minimum(g * BLK, LAST_BASE), 8)
            return g, base

        def fire_coords(s, base):
            pltpu.async_copy(b_hbm.at[pl.ds(base, BLK)], bv[s], sem_c[s])
            pltpu.async_copy(y_hbm.at[pl.ds(base, BLK)], yv[s], sem_c[s])
            pltpu.async_copy(x_hbm.at[pl.ds(base, BLK)], xv[s], sem_c[s])

        def wait_coords(s, base):
            pltpu.make_async_copy(
                b_hbm.at[pl.ds(base, BLK)], bv[s], sem_c[s]).wait()
            pltpu.make_async_copy(
                y_hbm.at[pl.ds(base, BLK)], yv[s], sem_c[s]).wait()
            pltpu.make_async_copy(
                x_hbm.at[pl.ds(base, BLK)], xv[s], sem_c[s]).wait()

        for s in range(2):
            g, base = blk_of(0, s)

            @pl.when(g < NBLK)
            def _(s=s, base=base):
                fire_coords(s, base)

        def body(jj, carry):
            for s in range(2):
                g, base = blk_of(jj, s)

                @pl.when(g < NBLK)
                def _(s=s, g=g, base=base):
                    wait_coords(s, base)
                    for i in range(BLK // 16):
                        sl = pl.ds(i * 16, 16)
                        idx[s][sl] = bv[s][sl] * HW + yv[s][sl] * W + xv[s][sl]

                    @pl.when(jj > 0)
                    def _():
                        prev = (g - 2 * NW) * BLK
                        pltpu.make_async_copy(
                            rows[s], out_hbm.at[pl.ds(prev, BLK)], sem_w[s]
                        ).wait()
                    pltpu.async_copy(table_hbm.at[idx[s]], rows[s], sem_g[s])

            for s in range(2):
                g, base = blk_of(jj, s)

                @pl.when(g < NBLK)
                def _(s=s, g=g, base=base):
                    pltpu.make_async_copy(
                        table_hbm.at[idx[s]], rows[s], sem_g[s]).wait()
                    pltpu.async_copy(
                        rows[s], out_hbm.at[pl.ds(base, BLK)], sem_w[s])
                    g2, base2 = blk_of(jj + 1, s)

                    @pl.when(g2 < NBLK)
                    def _():
                        fire_coords(s, base2)

            return carry

        lax.fori_loop(0, NITER, body, 0)
        for s in range(2):
            pltpu.make_async_copy(
                rows[s], out_hbm.at[pl.ds(0, BLK)], sem_w[s]).wait()

    return gather_kernel


_gather = _make_gather()


def kernel(spatial_features_2d, voxel_coords):
    table = jnp.transpose(spatial_features_2d, (0, 2, 3, 1)).reshape(B * HW, C)
    vc = voxel_coords.astype(jnp.int32)
    return _gather(table, vc[:, 0], vc[:, 2], vc[:, 3])


# speedup vs baseline: 1.0407x; 1.0407x over previous
import functools

import jax
import jax.numpy as jnp
from jax import lax
from jax.experimental import pallas as pl
from jax.experimental.pallas import tpu as pltpu
from jax.experimental.pallas import tpu_sc as plsc

B, C, H, W = 4, 256, 200, 176
HW = H * W
N = 100000
BLK = 192
NBLK = (N + BLK - 1) // BLK
LAST_BASE = N - BLK
NW = 32
NITER = ((NBLK + NW - 1) // NW + 1) // 2


def _make_gather():
    mesh = plsc.VectorSubcoreMesh(core_axis_name="c", subcore_axis_name="s")

    @functools.partial(
        pl.kernel,
        mesh=mesh,
        out_type=jax.ShapeDtypeStruct((N, C), jnp.float32),
        scratch_types=[
            pltpu.VMEM((BLK,), jnp.int32), pltpu.VMEM((BLK,), jnp.int32),
            pltpu.VMEM((BLK,), jnp.int32), pltpu.VMEM((BLK,), jnp.int32),
            pltpu.VMEM((BLK,), jnp.int32), pltpu.VMEM((BLK,), jnp.int32),
            pltpu.VMEM((BLK,), jnp.int32), pltpu.VMEM((BLK,), jnp.int32),
            pltpu.VMEM((BLK, C), jnp.float32),
            pltpu.VMEM((BLK, C), jnp.float32),
            pltpu.SemaphoreType.DMA, pltpu.SemaphoreType.DMA,
            pltpu.SemaphoreType.DMA, pltpu.SemaphoreType.DMA,
            pltpu.SemaphoreType.DMA, pltpu.SemaphoreType.DMA,
        ],
    )
    def gather_kernel(table_hbm, coords_flat_hbm, out_hbm,
                      bv0, yv0, xv0, bv1, yv1, xv1, idx0, idx1,
                      rows0, rows1,
                      sem_c0, sem_c1, sem_g0, sem_g1, sem_w0, sem_w1):
        wid = lax.axis_index("s") * 2 + lax.axis_index("c")
        bv, yv, xv = (bv0, bv1), (yv0, yv1), (xv0, xv1)
        idx, rows = (idx0, idx1), (rows0, rows1)
        sem_c, sem_g, sem_w = (sem_c0, sem_c1), (sem_g0, sem_g1), (sem_w0, sem_w1)

        def blk_of(jj, s):
            g = wid + NW * (2 * jj + s)
            base = pl.multiple_of(jnp.minimum(g * BLK, LAST_BASE), 8)
            return g, base

        def fire_coords(s, base):
            pltpu.async_copy(coords_flat_hbm.at[pl.ds(base, BLK)], bv[s], sem_c[s])
            pltpu.async_copy(coords_flat_hbm.at[pl.ds(2 * N + base, BLK)], yv[s], sem_c[s])
            pltpu.async_copy(coords_flat_hbm.at[pl.ds(3 * N + base, BLK)], xv[s], sem_c[s])

        def wait_coords(s, base):
            pltpu.make_async_copy(
                coords_flat_hbm.at[pl.ds(base, BLK)], bv[s], sem_c[s]).wait()
            pltpu.make_async_copy(
                coords_flat_hbm.at[pl.ds(2 * N + base, BLK)], yv[s], sem_c[s]).wait()
            pltpu.make_async_copy(
                coords_flat_hbm.at[pl.ds(3 * N + base, BLK)], xv[s], sem_c[s]).wait()

        for s in range(2):
            g, base = blk_of(0, s)

            @pl.when(g < NBLK)
            def _(s=s, base=base):
                fire_coords(s, base)

        def body(jj, carry):
            for s in range(2):
                g, base = blk_of(jj, s)

                @pl.when(g < NBLK)
                def _(s=s, g=g, base=base):
                    wait_coords(s, base)
                    for i in range(BLK // 16):
                        sl = pl.ds(i * 16, 16)
                        idx[s][sl] = bv[s][sl] * HW + yv[s][sl] * W + xv[s][sl]

                    @pl.when(jj > 0)
                    def _():
                        prev = (g - 2 * NW) * BLK
                        pltpu.make_async_copy(
                            rows[s], out_hbm.at[pl.ds(prev, BLK)], sem_w[s]
                        ).wait()
                    pltpu.async_copy(table_hbm.at[idx[s]], rows[s], sem_g[s])

            for s in range(2):
                g, base = blk_of(jj, s)

                @pl.when(g < NBLK)
                def _(s=s, g=g, base=base):
                    pltpu.make_async_copy(
                        table_hbm.at[idx[s]], rows[s], sem_g[s]).wait()
                    pltpu.async_copy(
                        rows[s], out_hbm.at[pl.ds(base, BLK)], sem_w[s])
                    g2, base2 = blk_of(jj + 1, s)

                    @pl.when(g2 < NBLK)
                    def _():
                        fire_coords(s, base2)

            return carry

        lax.fori_loop(0, NITER, body, 0)
        for s in range(2):
            pltpu.make_async_copy(
                rows[s], out_hbm.at[pl.ds(0, BLK)], sem_w[s]).wait()

    return gather_kernel


_gather = _make_gather()


def kernel(spatial_features_2d, voxel_coords):
    table = jnp.transpose(spatial_features_2d, (0, 2, 3, 1)).reshape(B * HW, C)
    vc_flat = jnp.transpose(voxel_coords.astype(jnp.int32), (1, 0)).reshape(4 * N)
    return _gather(table, vc_flat)
